# Initial kernel scaffold; baseline (speedup 1.0000x reference)
import jax
import jax.numpy as jnp
from jax.experimental import pallas as pl


def kernel(x, edge_index, edge_attr, graph_feat, batch, params):
    raise NotImplementedError("write your pallas kernel here")


# speedup vs baseline: 1.1320x; 1.1320x over previous
import functools

import jax
import jax.numpy as jnp
from jax import lax
from jax.experimental import pallas as pl
from jax.experimental.pallas import tpu as pltpu
from jax.experimental.pallas import tpu_sc as plsc

N_NODES = 10000
NSEG = 64
D = 128
NW = 32
CHUNK = 128
E_RAW = 320000
E_PAD = 323584
K_CH = E_PAD // (NW * CHUNK)
EPW = E_PAD // NW
NPAD = 10112
RPS = NPAD // 16

_INV_SQRT2 = 0.7071067811865476


def _gelu(x):
    return 0.5 * x * (1.0 + lax.erf(x * _INV_SQRT2))


def _ln(x, g, b, eps=1e-5):
    mu = jnp.mean(x, axis=-1, keepdims=True)
    var = jnp.mean((x - mu) ** 2, axis=-1, keepdims=True)
    return (x - mu) / jnp.sqrt(var + eps) * g + b


def _dot(a, b):
    return jax.lax.dot_general(a, b, (((1,), (0,)), ((), ())),
                               precision=jax.lax.Precision.HIGHEST,
                               preferred_element_type=jnp.float32)


def _sc_gather(table, idx3):
    mesh = plsc.VectorSubcoreMesh(core_axis_name="c", subcore_axis_name="s")

    @functools.partial(
        pl.kernel, mesh=mesh,
        out_type=jax.ShapeDtypeStruct((E_PAD, D), jnp.float32),
        scratch_types=[
            pltpu.VMEM((K_CH, CHUNK), jnp.int32),
            pltpu.VMEM((CHUNK, D), jnp.float32),
            pltpu.SemaphoreType.DMA,
        ],
    )
    def k(table_hbm, idx_hbm, out_hbm, idx_v, rows_v, sem):
        wid = lax.axis_index("s") * 2 + lax.axis_index("c")
        pltpu.sync_copy(idx_hbm.at[wid], idx_v)
        base = wid * EPW

        def step(j, carry):
            pltpu.async_copy(table_hbm.at[idx_v.at[j]], rows_v, sem).wait()
            pltpu.sync_copy(rows_v, out_hbm.at[pl.ds(base + j * CHUNK, CHUNK)])
            return carry

        lax.fori_loop(0, K_CH, step, 0)

    return k(table, idx3)


def _sc_scatter_add(msgs, idx3, zeros):
    mesh = plsc.VectorSubcoreMesh(core_axis_name="c", subcore_axis_name="s")

    @functools.partial(
        pl.kernel, mesh=mesh,
        out_type=jax.ShapeDtypeStruct((2, NPAD, D), jnp.float32),
        scratch_types=[
            pltpu.VMEM((K_CH, CHUNK), jnp.int32),
            pltpu.VMEM((CHUNK, D), jnp.float32),
            pltpu.VMEM_SHARED((NPAD, D), jnp.float32),
            pltpu.SemaphoreType.DMA,
        ],
    )
    def k(msgs_hbm, idx_hbm, zeros_hbm, out_hbm, idx_v, rows_v, acc_sh, sem):
        c = lax.axis_index("c")
        s = lax.axis_index("s")
        wid = s * 2 + c
        pltpu.sync_copy(zeros_hbm.at[pl.ds(s * RPS, RPS)],
                        acc_sh.at[pl.ds(s * RPS, RPS)])
        plsc.subcore_barrier()
        pltpu.sync_copy(idx_hbm.at[wid], idx_v)
        base = wid * EPW

        def step(j, carry):
            pltpu.sync_copy(msgs_hbm.at[pl.ds(base + j * CHUNK, CHUNK)], rows_v)
            pltpu.sync_copy(rows_v, acc_sh.at[idx_v.at[j]], add=True)
            return carry

        lax.fori_loop(0, K_CH, step, 0)
        plsc.subcore_barrier()
        pltpu.sync_copy(acc_sh.at[pl.ds(s * RPS, RPS)],
                        out_hbm.at[c, pl.ds(s * RPS, RPS)])

    return k(msgs, idx3, zeros)


def _node_embed(xp, wt, b, g, bb):
    BT = 1000

    def body(x_ref, wt_ref, b_ref, g_ref, bb_ref, o_ref):
        o_ref[...] = _gelu(_ln(_dot(x_ref[...], wt_ref[...]) + b_ref[...],
                               g_ref[...], bb_ref[...]))

    return pl.pallas_call(
        body,
        grid=(N_NODES // BT,),
        in_specs=[
            pl.BlockSpec((BT, 32), lambda i: (i, 0)),
            pl.BlockSpec((32, D), lambda i: (0, 0)),
            pl.BlockSpec((1, D), lambda i: (0, 0)),
            pl.BlockSpec((1, D), lambda i: (0, 0)),
            pl.BlockSpec((1, D), lambda i: (0, 0)),
        ],
        out_specs=pl.BlockSpec((BT, D), lambda i: (i, 0)),
        out_shape=jax.ShapeDtypeStruct((N_NODES, D), jnp.float32),
    )(xp, wt, b, g, bb)


def _edge_mlp(xj, ea, ewt, eb, w1xt, w1et, b1, g1, bb1, w2t, b2):
    BT = 1024

    def body(xj_ref, ea_ref, ewt_ref, eb_ref, w1xt_ref, w1et_ref, b1_ref,
             g1_ref, bb1_ref, w2t_ref, b2_ref, o_ref):
        e = _gelu(_dot(ea_ref[...], ewt_ref[...]) + eb_ref[...])
        m = (_dot(xj_ref[...], w1xt_ref[...]) + _dot(e, w1et_ref[...])
             + b1_ref[...])
        m = _gelu(_ln(m, g1_ref[...], bb1_ref[...]))
        o_ref[...] = _dot(m, w2t_ref[...]) + b2_ref[...]

    return pl.pallas_call(
        body,
        grid=(E_PAD // BT,),
        in_specs=[
            pl.BlockSpec((BT, D), lambda i: (i, 0)),
            pl.BlockSpec((BT, 8), lambda i: (i, 0)),
            pl.BlockSpec((8, 64), lambda i: (0, 0)),
            pl.BlockSpec((1, 64), lambda i: (0, 0)),
            pl.BlockSpec((D, D), lambda i: (0, 0)),
            pl.BlockSpec((64, D), lambda i: (0, 0)),
            pl.BlockSpec((1, D), lambda i: (0, 0)),
            pl.BlockSpec((1, D), lambda i: (0, 0)),
            pl.BlockSpec((1, D), lambda i: (0, 0)),
            pl.BlockSpec((D, D), lambda i: (0, 0)),
            pl.BlockSpec((1, D), lambda i: (0, 0)),
        ],
        out_specs=pl.BlockSpec((BT, D), lambda i: (i, 0)),
        out_shape=jax.ShapeDtypeStruct((E_PAD, D), jnp.float32),
    )(xj, ea, ewt, eb, w1xt, w1et, b1, g1, bb1, w2t, b2)


def _node_update(h, parts, w1ht, w1at, b1, ug, ub, w2t, b2, ng, nb):
    BT = 1000

    def body(h_ref, p_ref, w1ht_ref, w1at_ref, b1_ref, ug_ref, ub_ref,
             w2t_ref, b2_ref, ng_ref, nb_ref, o_ref):
        h_blk = h_ref[...]
        agg = p_ref[0] + p_ref[1]
        u = (_dot(h_blk, w1ht_ref[...]) + _dot(agg, w1at_ref[...])
             + b1_ref[...])
        u = _gelu(_ln(u, ug_ref[...], ub_ref[...]))
        u = _dot(u, w2t_ref[...]) + b2_ref[...]
        o_ref[...] = _ln(u + h_blk, ng_ref[...], nb_ref[...])

    return pl.pallas_call(
        body,
        grid=(N_NODES // BT,),
        in_specs=[
            pl.BlockSpec((BT, D), lambda i: (i, 0)),
            pl.BlockSpec((2, BT, D), lambda i: (0, i, 0)),
            pl.BlockSpec((D, D), lambda i: (0, 0)),
            pl.BlockSpec((D, D), lambda i: (0, 0)),
            pl.BlockSpec((1, D), lambda i: (0, 0)),
            pl.BlockSpec((1, D), lambda i: (0, 0)),
            pl.BlockSpec((1, D), lambda i: (0, 0)),
            pl.BlockSpec((D, D), lambda i: (0, 0)),
            pl.BlockSpec((1, D), lambda i: (0, 0)),
            pl.BlockSpec((1, D), lambda i: (0, 0)),
            pl.BlockSpec((1, D), lambda i: (0, 0)),
        ],
        out_specs=pl.BlockSpec((BT, D), lambda i: (i, 0)),
        out_shape=jax.ShapeDtypeStruct((N_NODES, D), jnp.float32),
    )(h, parts, w1ht, w1at, b1, ug, ub, w2t, b2, ng, nb)


def _pool_head(h, batch2, batchT, gf, a1t, a1b, a2t, a2b,
               w1at, w1bt, w1ct, w1dt, h1b, hg, hb, h2t, h2b,
               omt, omb, ovt, ovb):
    def body(h_ref, b2_ref, bT_ref, gf_ref, a1t_ref, a1b_ref, a2t_ref,
             a2b_ref, w1at_ref, w1bt_ref, w1ct_ref, w1dt_ref, h1b_ref,
             hg_ref, hb_ref, h2t_ref, h2b_ref, omt_ref, omb_ref,
             ovt_ref, ovb_ref, pm_ref, plv_ref, hmax_sc):
        h_all = h_ref[...]
        b_col = b2_ref[...]
        b_row = bT_ref[...]
        seg_row = lax.broadcasted_iota(jnp.int32, (1, NSEG), 1)
        seg_col = lax.broadcasted_iota(jnp.int32, (NSEG, 1), 0)
        P = b_col == seg_row
        Pf = P.astype(jnp.float32)
        PfT = (seg_col == b_row).astype(jnp.float32)

        s = jnp.tanh(_dot(h_all, a1t_ref[...]) + a1b_ref[...])
        sc = _dot(s, a2t_ref[...]) + a2b_ref[...]

        neg = jnp.float32(-jnp.inf)
        smax = jnp.max(jnp.where(P, sc, neg), axis=0, keepdims=True)
        smax = jnp.where(smax > neg, smax, 0.0)
        smax_row = jnp.sum(Pf * smax, axis=1, keepdims=True)
        ex = jnp.exp(sc - smax_row)
        denom = jnp.sum(Pf * ex, axis=0, keepdims=True)
        denom_row = jnp.sum(Pf * denom, axis=1, keepdims=True)
        w = ex / (denom_row + 1e-16)

        h_attn = _dot(PfT, w * h_all)
        counts = _dot(PfT, jnp.ones((N_NODES, 1), jnp.float32))
        h_mean = _dot(PfT, h_all) / jnp.maximum(counts, 1.0)

        def seg_step(i, carry):
            mask = b_col == i
            hm = jnp.max(jnp.where(mask, h_all, neg), axis=0,
                         keepdims=True)
            hmax_sc[pl.ds(i, 1), :] = hm
            return carry

        lax.fori_loop(0, NSEG, seg_step, 0)
        h_max = jnp.where(counts > 0.0, hmax_sc[...], 0.0)

        gf_v = gf_ref[...]
        gmu = jnp.mean(gf_v, axis=0, keepdims=True)
        gstd = jnp.sqrt(jnp.mean((gf_v - gmu) ** 2, axis=0, keepdims=True))
        gf_norm = (gf_v - gmu) / (gstd + 1e-6)

        z = (_dot(h_attn, w1at_ref[...]) + _dot(h_mean, w1bt_ref[...])
             + _dot(h_max, w1ct_ref[...]) + _dot(gf_norm, w1dt_ref[...])
             + h1b_ref[...])
        z = _gelu(_ln(z, hg_ref[...], hb_ref[...]))
        z = _gelu(_dot(z, h2t_ref[...]) + h2b_ref[...])
        pm_ref[...] = _dot(z, omt_ref[...]) + omb_ref[...]
        plv_ref[...] = _dot(z, ovt_ref[...]) + ovb_ref[...]

    full = lambda a: pl.BlockSpec(a.shape, lambda: tuple(0 for _ in a.shape))
    args = (h, batch2, batchT, gf, a1t, a1b, a2t, a2b, w1at, w1bt, w1ct,
            w1dt, h1b, hg, hb, h2t, h2b, omt, omb, ovt, ovb)
    return pl.pallas_call(
        body,
        in_specs=[full(a) for a in args],
        out_specs=(pl.BlockSpec((NSEG, 1), lambda: (0, 0)),
                   pl.BlockSpec((NSEG, 1), lambda: (0, 0))),
        out_shape=(jax.ShapeDtypeStruct((NSEG, 1), jnp.float32),
                   jax.ShapeDtypeStruct((NSEG, 1), jnp.float32)),
        scratch_shapes=[pltpu.VMEM((NSEG, D), jnp.float32)],
    )(*args)


def kernel(x, edge_index, edge_attr, graph_feat, batch, params):
    p = params
    f32 = jnp.float32
    r1 = lambda v: v.reshape(1, -1).astype(f32)

    xp = jnp.pad(x, ((0, 0), (0, 2)))
    node_wt = jnp.pad(p['node_w'], ((0, 0), (0, 2))).T

    src = edge_index[0]
    dst = edge_index[1]
    pad_e = E_PAD - E_RAW
    src3 = jnp.concatenate([src, jnp.zeros((pad_e,), jnp.int32)]
                           ).reshape(NW, K_CH, CHUNK)
    dst3 = jnp.concatenate([dst, jnp.full((pad_e,), N_NODES, jnp.int32)]
                           ).reshape(NW, K_CH, CHUNK)
    ea = jnp.pad(edge_attr, ((0, pad_e), (0, 0)))
    zeros_npad = jnp.zeros((NPAD, D), f32)

    h = _node_embed(xp, node_wt, r1(p['node_b']),
                    r1(p['node_ln_g']), r1(p['node_ln_b']))

    ewt = p['edge_w'].T
    eb = r1(p['edge_b'])

    for lp in p['layers']:
        xj = _sc_gather(h, src3)
        msgs = _edge_mlp(
            xj, ea, ewt, eb,
            lp['e1_w'][:, :D].T, lp['e1_w'][:, D:].T, r1(lp['e1_b']),
            r1(lp['e_ln_g']), r1(lp['e_ln_b']),
            lp['e2_w'].T, r1(lp['e2_b']))
        parts = _sc_scatter_add(msgs, dst3, zeros_npad)
        h = _node_update(
            h, parts,
            lp['u1_w'][:, :D].T, lp['u1_w'][:, D:].T, r1(lp['u1_b']),
            r1(lp['u_ln_g']), r1(lp['u_ln_b']),
            lp['u2_w'].T, r1(lp['u2_b']),
            r1(lp['n_g']), r1(lp['n_b']))

    h1w = p['h1_w']
    pm, plv = _pool_head(
        h, batch.reshape(N_NODES, 1), batch.reshape(1, N_NODES), graph_feat,
        p['attn1_w'].T, r1(p['attn1_b']), p['attn2_w'].T, r1(p['attn2_b']),
        h1w[:, :D].T, h1w[:, D:2 * D].T, h1w[:, 2 * D:3 * D].T,
        h1w[:, 3 * D:].T, r1(p['h1_b']),
        r1(p['h_ln_g']), r1(p['h_ln_b']),
        p['h2_w'].T, r1(p['h2_b']),
        p['om_w'].T, r1(p['om_b']), p['ov_w'].T, r1(p['ov_b']))
    return (pm, plv)


# speedup vs baseline: 1.8727x; 1.6543x over previous
import functools

import jax
import jax.numpy as jnp
from jax import lax
from jax.experimental import pallas as pl
from jax.experimental.pallas import tpu as pltpu
from jax.experimental.pallas import tpu_sc as plsc

N_NODES = 10000
NSEG = 64
D = 128
NW = 32
CHUNK = 128
E_RAW = 320000
E_PAD = 327680
K_CH = E_PAD // (NW * CHUNK)
EPW = E_PAD // NW
NPAD = 10112
RPS = NPAD // 16

_SQRT2 = 1.4142135623730951


def _gelu(x):
    return x * (1.0 + lax.erf(x / _SQRT2)) / 2.0


def _ln(x, g, b, eps=1e-5):
    mu = jnp.mean(x, axis=-1, keepdims=True)
    var = jnp.mean((x - mu) ** 2, axis=-1, keepdims=True)
    return (x - mu) / jnp.sqrt(var + eps) * g + b


def _dot(a, b):
    return jax.lax.dot_general(a.astype(jnp.bfloat16),
                               b.astype(jnp.bfloat16),
                               (((1,), (0,)), ((), ())),
                               preferred_element_type=jnp.float32)


def _dot_exact(a, b):
    return jax.lax.dot_general(a, b, (((1,), (0,)), ((), ())),
                               precision=jax.lax.Precision.HIGHEST,
                               preferred_element_type=jnp.float32)


def _sc_gather(table, idx3):
    mesh = plsc.VectorSubcoreMesh(core_axis_name="c", subcore_axis_name="s")

    @functools.partial(
        pl.kernel, mesh=mesh,
        out_type=jax.ShapeDtypeStruct((E_PAD, D), jnp.float32),
        scratch_types=[
            pltpu.VMEM((K_CH, CHUNK), jnp.int32),
            pltpu.VMEM((2, CHUNK, D), jnp.float32),
            pltpu.SemaphoreType.DMA,
            pltpu.SemaphoreType.DMA,
        ],
    )
    def k(table_hbm, idx_hbm, out_hbm, idx_v, rows_v, sema, semb):
        wid = lax.axis_index("s") * 2 + lax.axis_index("c")
        pltpu.sync_copy(idx_hbm.at[wid], idx_v)
        base = wid * EPW

        def step(j, carry):
            pltpu.async_copy(table_hbm.at[idx_v.at[j]], rows_v.at[0],
                             sema).wait()
            pltpu.sync_copy(rows_v.at[0],
                            out_hbm.at[pl.ds(base + j * CHUNK, CHUNK)])
            return carry

        lax.fori_loop(0, K_CH, step, 0)

    return k(table, idx3)


def _sc_scatter_add(msgs, idx3, zeros):
    mesh = plsc.VectorSubcoreMesh(core_axis_name="c", subcore_axis_name="s")

    @functools.partial(
        pl.kernel, mesh=mesh,
        out_type=jax.ShapeDtypeStruct((2, NPAD, D), jnp.float32),
        scratch_types=[
            pltpu.VMEM((K_CH, CHUNK), jnp.int32),
            pltpu.VMEM((2, CHUNK, D), jnp.float32),
            pltpu.VMEM_SHARED((NPAD, D), jnp.float32),
            pltpu.SemaphoreType.DMA,
            pltpu.SemaphoreType.DMA,
        ],
    )
    def k(msgs_hbm, idx_hbm, zeros_hbm, out_hbm, idx_v, rows_v, acc_sh,
          sema, semb):
        c = lax.axis_index("c")
        s = lax.axis_index("s")
        wid = s * 2 + c
        base = wid * EPW
        pltpu.sync_copy(idx_hbm.at[wid], idx_v)
        pltpu.sync_copy(zeros_hbm.at[pl.ds(s * RPS, RPS)],
                        acc_sh.at[pl.ds(s * RPS, RPS)])
        plsc.subcore_barrier()

        def step(j, carry):
            pltpu.async_copy(msgs_hbm.at[pl.ds(base + j * CHUNK, CHUNK)],
                             rows_v.at[0], sema).wait()
            pltpu.sync_copy(rows_v.at[0], acc_sh.at[idx_v.at[j]], add=True)
            return carry

        lax.fori_loop(0, K_CH, step, 0)
        plsc.subcore_barrier()
        pltpu.sync_copy(acc_sh.at[pl.ds(s * RPS, RPS)],
                        out_hbm.at[c, pl.ds(s * RPS, RPS)])

    return k(msgs, idx3, zeros)


def _node_embed(xp, wt, b, g, bb):
    BT = 1000

    def body(x_ref, wt_ref, b_ref, g_ref, bb_ref, o_ref):
        o_ref[...] = _gelu(_ln(_dot(x_ref[...], wt_ref[...]) + b_ref[...],
                               g_ref[...], bb_ref[...]))

    return pl.pallas_call(
        body,
        grid=(N_NODES // BT,),
        in_specs=[
            pl.BlockSpec((BT, 32), lambda i: (i, 0)),
            pl.BlockSpec((32, D), lambda i: (0, 0)),
            pl.BlockSpec((1, D), lambda i: (0, 0)),
            pl.BlockSpec((1, D), lambda i: (0, 0)),
            pl.BlockSpec((1, D), lambda i: (0, 0)),
        ],
        out_specs=pl.BlockSpec((BT, D), lambda i: (i, 0)),
        out_shape=jax.ShapeDtypeStruct((N_NODES, D), jnp.float32),
    )(xp, wt, b, g, bb)


def _edge_mlp(xj, ea, ewt, eb, w1t, b1, g1, bb1, w2t, b2):
    BT = 2048

    def body(xj_ref, ea_ref, ewt_ref, eb_ref, w1t_ref, b1_ref, g1_ref,
             bb1_ref, w2t_ref, b2_ref, o_ref):
        e = _gelu(_dot(ea_ref[...], ewt_ref[...]) + eb_ref[...])
        m = jnp.concatenate([xj_ref[...], e], axis=-1)
        m = _dot(m, w1t_ref[...]) + b1_ref[...]
        m = _gelu(_ln(m, g1_ref[...], bb1_ref[...]))
        o_ref[...] = _dot(m, w2t_ref[...]) + b2_ref[...]

    return pl.pallas_call(
        body,
        grid=(E_PAD // BT,),
        in_specs=[
            pl.BlockSpec((BT, D), lambda i: (i, 0)),
            pl.BlockSpec((BT, 8), lambda i: (i, 0)),
            pl.BlockSpec((8, 64), lambda i: (0, 0)),
            pl.BlockSpec((1, 64), lambda i: (0, 0)),
            pl.BlockSpec((192, D), lambda i: (0, 0)),
            pl.BlockSpec((1, D), lambda i: (0, 0)),
            pl.BlockSpec((1, D), lambda i: (0, 0)),
            pl.BlockSpec((1, D), lambda i: (0, 0)),
            pl.BlockSpec((D, D), lambda i: (0, 0)),
            pl.BlockSpec((1, D), lambda i: (0, 0)),
        ],
        out_specs=pl.BlockSpec((BT, D), lambda i: (i, 0)),
        out_shape=jax.ShapeDtypeStruct((E_PAD, D), jnp.float32),
    )(xj, ea, ewt, eb, w1t, b1, g1, bb1, w2t, b2)


def _node_update(h, parts, w1t, b1, ug, ub, w2t, b2, ng, nb):
    BT = 1000

    def body(h_ref, p_ref, w1t_ref, b1_ref, ug_ref, ub_ref,
             w2t_ref, b2_ref, ng_ref, nb_ref, o_ref):
        h_blk = h_ref[...]
        agg = p_ref[0] + p_ref[1]
        u = jnp.concatenate([h_blk, agg], axis=-1)
        u = _dot(u, w1t_ref[...]) + b1_ref[...]
        u = _gelu(_ln(u, ug_ref[...], ub_ref[...]))
        u = _dot(u, w2t_ref[...]) + b2_ref[...]
        o_ref[...] = _ln(u + h_blk, ng_ref[...], nb_ref[...])

    return pl.pallas_call(
        body,
        grid=(N_NODES // BT,),
        in_specs=[
            pl.BlockSpec((BT, D), lambda i: (i, 0)),
            pl.BlockSpec((2, BT, D), lambda i: (0, i, 0)),
            pl.BlockSpec((256, D), lambda i: (0, 0)),
            pl.BlockSpec((1, D), lambda i: (0, 0)),
            pl.BlockSpec((1, D), lambda i: (0, 0)),
            pl.BlockSpec((1, D), lambda i: (0, 0)),
            pl.BlockSpec((D, D), lambda i: (0, 0)),
            pl.BlockSpec((1, D), lambda i: (0, 0)),
            pl.BlockSpec((1, D), lambda i: (0, 0)),
            pl.BlockSpec((1, D), lambda i: (0, 0)),
        ],
        out_specs=pl.BlockSpec((BT, D), lambda i: (i, 0)),
        out_shape=jax.ShapeDtypeStruct((N_NODES, D), jnp.float32),
    )(h, parts, w1t, b1, ug, ub, w2t, b2, ng, nb)


def _pool_head(h, batch2, batchT, gf, a1t, a1b, a2t, a2b,
               w1t, h1b, hg, hb, h2t, h2b,
               omt, omb, ovt, ovb):
    def body(h_ref, b2_ref, bT_ref, gf_ref, a1t_ref, a1b_ref, a2t_ref,
             a2b_ref, w1t_ref, h1b_ref,
             hg_ref, hb_ref, h2t_ref, h2b_ref, omt_ref, omb_ref,
             ovt_ref, ovb_ref, pm_ref, plv_ref, hmax_sc):
        h_all = h_ref[...]
        b_col = b2_ref[...]
        b_row = bT_ref[...]
        seg_row = lax.broadcasted_iota(jnp.int32, (1, NSEG), 1)
        seg_col = lax.broadcasted_iota(jnp.int32, (NSEG, 1), 0)
        P = b_col == seg_row
        Pf = P.astype(jnp.float32)
        PfT = (seg_col == b_row).astype(jnp.float32)

        s = jnp.tanh(_dot(h_all, a1t_ref[...]) + a1b_ref[...])
        sc = _dot(s, a2t_ref[...]) + a2b_ref[...]

        neg = jnp.float32(-jnp.inf)
        smax = jnp.max(jnp.where(P, sc, neg), axis=0, keepdims=True)
        smax = jnp.where(smax > neg, smax, 0.0)
        smax_row = jnp.sum(Pf * smax, axis=1, keepdims=True)
        ex = jnp.exp(sc - smax_row)
        denom = jnp.sum(Pf * ex, axis=0, keepdims=True)
        denom_row = jnp.sum(Pf * denom, axis=1, keepdims=True)
        w = ex / (denom_row + 1e-16)

        h_attn = _dot_exact(PfT, w * h_all)
        counts = _dot_exact(PfT, jnp.ones((N_NODES, 1), jnp.float32))
        h_mean = _dot_exact(PfT, h_all) / jnp.maximum(counts, 1.0)

        def seg_step(i, carry):
            mask = b_col == i
            hm = jnp.max(jnp.where(mask, h_all, neg), axis=0,
                         keepdims=True)
            hmax_sc[pl.ds(i, 1), :] = hm
            return carry

        lax.fori_loop(0, NSEG, seg_step, 0)
        h_max = jnp.where(counts > 0.0, hmax_sc[...], 0.0)

        gf_v = gf_ref[...]
        gmu = jnp.mean(gf_v, axis=0, keepdims=True)
        gstd = jnp.sqrt(jnp.mean((gf_v - gmu) ** 2, axis=0, keepdims=True))
        gf_norm = (gf_v - gmu) / (gstd + 1e-6)

        pooled = jnp.concatenate([h_attn, h_mean, h_max, gf_norm], axis=-1)
        z = _dot(pooled, w1t_ref[...]) + h1b_ref[...]
        z = _gelu(_ln(z, hg_ref[...], hb_ref[...]))
        z = _gelu(_dot(z, h2t_ref[...]) + h2b_ref[...])
        pm_ref[...] = _dot(z, omt_ref[...]) + omb_ref[...]
        plv_ref[...] = _dot(z, ovt_ref[...]) + ovb_ref[...]

    full = lambda a: pl.BlockSpec(a.shape, lambda: tuple(0 for _ in a.shape))
    args = (h, batch2, batchT, gf, a1t, a1b, a2t, a2b, w1t,
            h1b, hg, hb, h2t, h2b, omt, omb, ovt, ovb)
    return pl.pallas_call(
        body,
        in_specs=[full(a) for a in args],
        out_specs=(pl.BlockSpec((NSEG, 1), lambda: (0, 0)),
                   pl.BlockSpec((NSEG, 1), lambda: (0, 0))),
        out_shape=(jax.ShapeDtypeStruct((NSEG, 1), jnp.float32),
                   jax.ShapeDtypeStruct((NSEG, 1), jnp.float32)),
        scratch_shapes=[pltpu.VMEM((NSEG, D), jnp.float32)],
    )(*args)


def kernel(x, edge_index, edge_attr, graph_feat, batch, params):
    p = params
    f32 = jnp.float32
    r1 = lambda v: v.reshape(1, -1).astype(f32)

    xp = jnp.pad(x, ((0, 0), (0, 2)))
    node_wt = jnp.pad(p['node_w'], ((0, 0), (0, 2))).T

    src = edge_index[0]
    dst = edge_index[1]
    pad_e = E_PAD - E_RAW
    src3 = jnp.concatenate([src, jnp.zeros((pad_e,), jnp.int32)]
                           ).reshape(NW, K_CH, CHUNK)
    dst3 = jnp.concatenate([dst, jnp.full((pad_e,), N_NODES, jnp.int32)]
                           ).reshape(NW, K_CH, CHUNK)
    ea = jnp.pad(edge_attr, ((0, pad_e), (0, 0)))
    zeros_npad = jnp.zeros((NPAD, D), f32)

    h = _node_embed(xp, node_wt, r1(p['node_b']),
                    r1(p['node_ln_g']), r1(p['node_ln_b']))

    ewt = p['edge_w'].T
    eb = r1(p['edge_b'])

    for lp in p['layers']:
        xj = _sc_gather(h, src3)
        msgs = _edge_mlp(xj, ea, ewt, eb, lp['e1_w'].T, r1(lp['e1_b']),
                         r1(lp['e_ln_g']), r1(lp['e_ln_b']),
                         lp['e2_w'].T, r1(lp['e2_b']))
        parts = _sc_scatter_add(msgs, dst3, zeros_npad)
        h = _node_update(
            h, parts, lp['u1_w'].T, r1(lp['u1_b']),
            r1(lp['u_ln_g']), r1(lp['u_ln_b']),
            lp['u2_w'].T, r1(lp['u2_b']),
            r1(lp['n_g']), r1(lp['n_b']))

    pm, plv = _pool_head(
        h, batch.reshape(N_NODES, 1), batch.reshape(1, N_NODES), graph_feat,
        p['attn1_w'].T, r1(p['attn1_b']), p['attn2_w'].T, r1(p['attn2_b']),
        p['h1_w'].T, r1(p['h1_b']),
        r1(p['h_ln_g']), r1(p['h_ln_b']),
        p['h2_w'].T, r1(p['h2_b']),
        p['om_w'].T, r1(p['om_b']), p['ov_w'].T, r1(p['ov_b']))
    return (pm, plv)


# speedup vs baseline: 1.8769x; 1.0022x over previous
import functools

import jax
import jax.numpy as jnp
from jax import lax
from jax.experimental import pallas as pl
from jax.experimental.pallas import tpu as pltpu
from jax.experimental.pallas import tpu_sc as plsc

N_NODES = 10000
NSEG = 64
D = 128
NW = 32
CHUNK = 128
E_RAW = 320000
E_PAD = 327680
K_CH = E_PAD // (NW * CHUNK)
EPW = E_PAD // NW
NPAD = 10112
RPS = NPAD // 16

_SQRT2 = 1.4142135623730951


def _gelu(x):
    return x * (1.0 + lax.erf(x / _SQRT2)) / 2.0


def _ln(x, g, b, eps=1e-5):
    mu = jnp.mean(x, axis=-1, keepdims=True)
    var = jnp.mean((x - mu) ** 2, axis=-1, keepdims=True)
    return (x - mu) / jnp.sqrt(var + eps) * g + b


def _dot(a, b):
    return jax.lax.dot_general(a.astype(jnp.bfloat16),
                               b.astype(jnp.bfloat16),
                               (((1,), (0,)), ((), ())),
                               preferred_element_type=jnp.float32)


def _dot_exact(a, b):
    return jax.lax.dot_general(a, b, (((1,), (0,)), ((), ())),
                               precision=jax.lax.Precision.HIGHEST,
                               preferred_element_type=jnp.float32)


def _sc_gather(table, idx3):
    mesh = plsc.VectorSubcoreMesh(core_axis_name="c", subcore_axis_name="s")

    @functools.partial(
        pl.kernel, mesh=mesh,
        out_type=jax.ShapeDtypeStruct((E_PAD, D), jnp.float32),
        scratch_types=[
            pltpu.VMEM((K_CH, CHUNK), jnp.int32),
            pltpu.VMEM((CHUNK, D), jnp.float32),
            pltpu.SemaphoreType.DMA,
        ],
    )
    def k(table_hbm, idx_hbm, out_hbm, idx_v, rows_v, sem):
        wid = lax.axis_index("s") * 2 + lax.axis_index("c")
        pltpu.sync_copy(idx_hbm.at[wid], idx_v)
        base = wid * EPW

        def step(j, carry):
            pltpu.async_copy(table_hbm.at[idx_v.at[j]], rows_v, sem).wait()
            pltpu.sync_copy(rows_v, out_hbm.at[pl.ds(base + j * CHUNK, CHUNK)])
            return carry

        lax.fori_loop(0, K_CH, step, 0)

    return k(table, idx3)


def _sc_scatter_add(msgs, idx3, zeros):
    mesh = plsc.VectorSubcoreMesh(core_axis_name="c", subcore_axis_name="s")

    @functools.partial(
        pl.kernel, mesh=mesh,
        out_type=jax.ShapeDtypeStruct((2, NPAD, D), jnp.float32),
        scratch_types=[
            pltpu.VMEM((K_CH, CHUNK), jnp.int32),
            pltpu.VMEM((CHUNK, D), jnp.float32),
            pltpu.VMEM_SHARED((NPAD, D), jnp.float32),
            pltpu.SemaphoreType.DMA,
        ],
    )
    def k(msgs_hbm, idx_hbm, zeros_hbm, out_hbm, idx_v, rows_v, acc_sh, sem):
        c = lax.axis_index("c")
        s = lax.axis_index("s")
        wid = s * 2 + c
        base = wid * EPW
        pltpu.sync_copy(idx_hbm.at[wid], idx_v)
        pltpu.sync_copy(zeros_hbm.at[pl.ds(s * RPS, RPS)],
                        acc_sh.at[pl.ds(s * RPS, RPS)])
        plsc.subcore_barrier()

        def step(j, carry):
            pltpu.async_copy(msgs_hbm.at[pl.ds(base + j * CHUNK, CHUNK)],
                             rows_v, sem).wait()
            pltpu.sync_copy(rows_v, acc_sh.at[idx_v.at[j]], add=True)
            return carry

        lax.fori_loop(0, K_CH, step, 0)
        plsc.subcore_barrier()
        pltpu.sync_copy(acc_sh.at[pl.ds(s * RPS, RPS)],
                        out_hbm.at[c, pl.ds(s * RPS, RPS)])

    return k(msgs, idx3, zeros)


def _node_embed(xp, wt, b, g, bb):
    BT = 1000

    def body(x_ref, wt_ref, b_ref, g_ref, bb_ref, o_ref):
        o_ref[...] = _gelu(_ln(_dot(x_ref[...], wt_ref[...]) + b_ref[...],
                               g_ref[...], bb_ref[...]))

    return pl.pallas_call(
        body,
        grid=(N_NODES // BT,),
        in_specs=[
            pl.BlockSpec((BT, 32), lambda i: (i, 0)),
            pl.BlockSpec((32, D), lambda i: (0, 0)),
            pl.BlockSpec((1, D), lambda i: (0, 0)),
            pl.BlockSpec((1, D), lambda i: (0, 0)),
            pl.BlockSpec((1, D), lambda i: (0, 0)),
        ],
        out_specs=pl.BlockSpec((BT, D), lambda i: (i, 0)),
        out_shape=jax.ShapeDtypeStruct((N_NODES, D), jnp.float32),
    )(xp, wt, b, g, bb)


def _edge_mlp(xj, ea, ewt, eb, w1t, b1, g1, bb1, w2t, b2):
    BT = 2048

    def body(xj_ref, ea_ref, ewt_ref, eb_ref, w1t_ref, b1_ref, g1_ref,
             bb1_ref, w2t_ref, b2_ref, o_ref):
        e = _gelu(_dot(ea_ref[...], ewt_ref[...]) + eb_ref[...])
        m = jnp.concatenate([xj_ref[...], e], axis=-1)
        m = _dot(m, w1t_ref[...]) + b1_ref[...]
        m = _gelu(_ln(m, g1_ref[...], bb1_ref[...]))
        o_ref[...] = _dot(m, w2t_ref[...]) + b2_ref[...]

    return pl.pallas_call(
        body,
        grid=(E_PAD // BT,),
        in_specs=[
            pl.BlockSpec((BT, D), lambda i: (i, 0)),
            pl.BlockSpec((BT, 8), lambda i: (i, 0)),
            pl.BlockSpec((8, 64), lambda i: (0, 0)),
            pl.BlockSpec((1, 64), lambda i: (0, 0)),
            pl.BlockSpec((192, D), lambda i: (0, 0)),
            pl.BlockSpec((1, D), lambda i: (0, 0)),
            pl.BlockSpec((1, D), lambda i: (0, 0)),
            pl.BlockSpec((1, D), lambda i: (0, 0)),
            pl.BlockSpec((D, D), lambda i: (0, 0)),
            pl.BlockSpec((1, D), lambda i: (0, 0)),
        ],
        out_specs=pl.BlockSpec((BT, D), lambda i: (i, 0)),
        out_shape=jax.ShapeDtypeStruct((E_PAD, D), jnp.float32),
    )(xj, ea, ewt, eb, w1t, b1, g1, bb1, w2t, b2)


def _node_update(h, parts, w1t, b1, ug, ub, w2t, b2, ng, nb):
    BT = 1000

    def body(h_ref, p_ref, w1t_ref, b1_ref, ug_ref, ub_ref,
             w2t_ref, b2_ref, ng_ref, nb_ref, o_ref):
        h_blk = h_ref[...]
        agg = p_ref[0] + p_ref[1]
        u = jnp.concatenate([h_blk, agg], axis=-1)
        u = _dot(u, w1t_ref[...]) + b1_ref[...]
        u = _gelu(_ln(u, ug_ref[...], ub_ref[...]))
        u = _dot(u, w2t_ref[...]) + b2_ref[...]
        o_ref[...] = _ln(u + h_blk, ng_ref[...], nb_ref[...])

    return pl.pallas_call(
        body,
        grid=(N_NODES // BT,),
        in_specs=[
            pl.BlockSpec((BT, D), lambda i: (i, 0)),
            pl.BlockSpec((2, BT, D), lambda i: (0, i, 0)),
            pl.BlockSpec((256, D), lambda i: (0, 0)),
            pl.BlockSpec((1, D), lambda i: (0, 0)),
            pl.BlockSpec((1, D), lambda i: (0, 0)),
            pl.BlockSpec((1, D), lambda i: (0, 0)),
            pl.BlockSpec((D, D), lambda i: (0, 0)),
            pl.BlockSpec((1, D), lambda i: (0, 0)),
            pl.BlockSpec((1, D), lambda i: (0, 0)),
            pl.BlockSpec((1, D), lambda i: (0, 0)),
        ],
        out_specs=pl.BlockSpec((BT, D), lambda i: (i, 0)),
        out_shape=jax.ShapeDtypeStruct((N_NODES, D), jnp.float32),
    )(h, parts, w1t, b1, ug, ub, w2t, b2, ng, nb)


def _pool_head(h, batch2, batchT, gf, a1t, a1b, a2t, a2b,
               w1t, h1b, hg, hb, h2t, h2b,
               omt, omb, ovt, ovb):
    def body(h_ref, b2_ref, bT_ref, gf_ref, a1t_ref, a1b_ref, a2t_ref,
             a2b_ref, w1t_ref, h1b_ref,
             hg_ref, hb_ref, h2t_ref, h2b_ref, omt_ref, omb_ref,
             ovt_ref, ovb_ref, pm_ref, plv_ref, hmax_sc):
        h_all = h_ref[...]
        b_col = b2_ref[...]
        b_row = bT_ref[...]
        seg_row = lax.broadcasted_iota(jnp.int32, (1, NSEG), 1)
        seg_col = lax.broadcasted_iota(jnp.int32, (NSEG, 1), 0)
        P = b_col == seg_row
        Pf = P.astype(jnp.float32)
        PfT = (seg_col == b_row).astype(jnp.float32)

        s = jnp.tanh(_dot(h_all, a1t_ref[...]) + a1b_ref[...])
        sc = _dot(s, a2t_ref[...]) + a2b_ref[...]

        neg = jnp.float32(-jnp.inf)
        smax = jnp.max(jnp.where(P, sc, neg), axis=0, keepdims=True)
        smax = jnp.where(smax > neg, smax, 0.0)
        smax_row = jnp.sum(Pf * smax, axis=1, keepdims=True)
        ex = jnp.exp(sc - smax_row)
        denom = jnp.sum(Pf * ex, axis=0, keepdims=True)
        denom_row = jnp.sum(Pf * denom, axis=1, keepdims=True)
        w = ex / (denom_row + 1e-16)

        h_attn = _dot_exact(PfT, w * h_all)
        counts = _dot_exact(PfT, jnp.ones((N_NODES, 1), jnp.float32))
        h_mean = _dot_exact(PfT, h_all) / jnp.maximum(counts, 1.0)

        def seg_step(i, carry):
            mask = b_col == i
            hm = jnp.max(jnp.where(mask, h_all, neg), axis=0,
                         keepdims=True)
            hmax_sc[pl.ds(i, 1), :] = hm
            return carry

        lax.fori_loop(0, NSEG, seg_step, 0)
        h_max = jnp.where(counts > 0.0, hmax_sc[...], 0.0)

        gf_v = gf_ref[...]
        gmu = jnp.mean(gf_v, axis=0, keepdims=True)
        gstd = jnp.sqrt(jnp.mean((gf_v - gmu) ** 2, axis=0, keepdims=True))
        gf_norm = (gf_v - gmu) / (gstd + 1e-6)

        pooled = jnp.concatenate([h_attn, h_mean, h_max, gf_norm], axis=-1)
        z = _dot(pooled, w1t_ref[...]) + h1b_ref[...]
        z = _gelu(_ln(z, hg_ref[...], hb_ref[...]))
        z = _gelu(_dot(z, h2t_ref[...]) + h2b_ref[...])
        pm_ref[...] = _dot(z, omt_ref[...]) + omb_ref[...]
        plv_ref[...] = _dot(z, ovt_ref[...]) + ovb_ref[...]

    full = lambda a: pl.BlockSpec(a.shape, lambda: tuple(0 for _ in a.shape))
    args = (h, batch2, batchT, gf, a1t, a1b, a2t, a2b, w1t,
            h1b, hg, hb, h2t, h2b, omt, omb, ovt, ovb)
    return pl.pallas_call(
        body,
        in_specs=[full(a) for a in args],
        out_specs=(pl.BlockSpec((NSEG, 1), lambda: (0, 0)),
                   pl.BlockSpec((NSEG, 1), lambda: (0, 0))),
        out_shape=(jax.ShapeDtypeStruct((NSEG, 1), jnp.float32),
                   jax.ShapeDtypeStruct((NSEG, 1), jnp.float32)),
        scratch_shapes=[pltpu.VMEM((NSEG, D), jnp.float32)],
    )(*args)


def kernel(x, edge_index, edge_attr, graph_feat, batch, params):
    p = params
    f32 = jnp.float32
    r1 = lambda v: v.reshape(1, -1).astype(f32)

    xp = jnp.pad(x, ((0, 0), (0, 2)))
    node_wt = jnp.pad(p['node_w'], ((0, 0), (0, 2))).T

    src = edge_index[0]
    dst = edge_index[1]
    pad_e = E_PAD - E_RAW
    src3 = jnp.concatenate([src, jnp.zeros((pad_e,), jnp.int32)]
                           ).reshape(NW, K_CH, CHUNK)
    dst3 = jnp.concatenate([dst, jnp.full((pad_e,), N_NODES, jnp.int32)]
                           ).reshape(NW, K_CH, CHUNK)
    ea = jnp.pad(edge_attr, ((0, pad_e), (0, 0)))
    zeros_npad = jnp.zeros((NPAD, D), f32)

    h = _node_embed(xp, node_wt, r1(p['node_b']),
                    r1(p['node_ln_g']), r1(p['node_ln_b']))

    ewt = p['edge_w'].T
    eb = r1(p['edge_b'])

    for lp in p['layers']:
        xj = _sc_gather(h, src3)
        msgs = _edge_mlp(xj, ea, ewt, eb, lp['e1_w'].T, r1(lp['e1_b']),
                         r1(lp['e_ln_g']), r1(lp['e_ln_b']),
                         lp['e2_w'].T, r1(lp['e2_b']))
        parts = _sc_scatter_add(msgs, dst3, zeros_npad)
        h = _node_update(
            h, parts, lp['u1_w'].T, r1(lp['u1_b']),
            r1(lp['u_ln_g']), r1(lp['u_ln_b']),
            lp['u2_w'].T, r1(lp['u2_b']),
            r1(lp['n_g']), r1(lp['n_b']))

    pm, plv = _pool_head(
        h, batch.reshape(N_NODES, 1), batch.reshape(1, N_NODES), graph_feat,
        p['attn1_w'].T, r1(p['attn1_b']), p['attn2_w'].T, r1(p['attn2_b']),
        p['h1_w'].T, r1(p['h1_b']),
        r1(p['h_ln_g']), r1(p['h_ln_b']),
        p['h2_w'].T, r1(p['h2_b']),
        p['om_w'].T, r1(p['om_b']), p['ov_w'].T, r1(p['ov_b']))
    return (pm, plv)
